# baseline (speedup 1.0000x reference)
import functools
import math

import jax
import jax.numpy as jnp
from jax import lax
from jax.experimental import pallas as pl
from jax.experimental.pallas import tpu as pltpu
from jax.experimental.pallas import tpu_sc as plsc

_NC = 2
_NS = 16
_NW = _NC * _NS
_C = 128
_M = 5
_K = 3


def _scale_body(t_ref, o_ref, *, scale):
    o_ref[...] = t_ref[...] * scale


def _scale_table(table, scale):
    v, d = table.shape
    br = 2000
    assert v % br == 0
    return pl.pallas_call(
        functools.partial(_scale_body, scale=scale),
        grid=(v // br,),
        in_specs=[pl.BlockSpec((br, d), lambda i: (i, 0))],
        out_specs=pl.BlockSpec((br, d), lambda i: (i, 0)),
        out_shape=jax.ShapeDtypeStruct((v, d), table.dtype),
    )(table)


def _make_gather(b_total, d):
    per_w = b_total // _NW
    n_chunks = per_w // _C
    assert n_chunks % _M == 0 and n_chunks > _M
    mesh = plsc.VectorSubcoreMesh(core_axis_name="c", subcore_axis_name="s")

    @functools.partial(
        pl.kernel,
        out_type=jax.ShapeDtypeStruct((b_total, d), jnp.float32),
        mesh=mesh,
        scratch_types=[
            pltpu.VMEM((n_chunks, _C), jnp.int32),
            pltpu.VMEM((_M, _C, d), jnp.float32),
        ] + [pltpu.SemaphoreType.DMA] * (2 * _M),
    )
    def gather(tab_hbm, idx_hbm, out_hbm, idx_v, rows_v, *sems):
        gsems, ssems = sems[:_M], sems[_M:]
        wid = lax.axis_index("s") * _NC + lax.axis_index("c")
        pltpu.sync_copy(idx_hbm.at[wid], idx_v)
        base = wid * per_w

        def g_start(c, b):
            pltpu.async_copy(tab_hbm.at[idx_v.at[c]], rows_v.at[b], gsems[b])

        def g_wait(c, b):
            pltpu.make_async_copy(
                tab_hbm.at[idx_v.at[c]], rows_v.at[b], gsems[b]).wait()

        def s_start(c, b):
            pltpu.async_copy(rows_v.at[b],
                             out_hbm.at[pl.ds(base + c * _C, _C)], ssems[b])

        def s_wait(c, b):
            pltpu.make_async_copy(
                rows_v.at[b],
                out_hbm.at[pl.ds(base + c * _C, _C)], ssems[b]).wait()

        for b in range(_K):
            g_start(b, b)

        def group(g, carry):
            for r in range(_M):
                c = g * _M + r
                g_wait(c, r)
                s_start(c, r)
                bp = (r + _K) % _M

                @pl.when(c >= _M - _K)
                def _():
                    s_wait(c + _K - _M, bp)

                @pl.when(c + _K < n_chunks)
                def _():
                    g_start(c + _K, bp)
            return carry

        lax.fori_loop(0, n_chunks // _M, group, 0)

        for c in range(n_chunks - (_M - _K), n_chunks):
            s_wait(c, c % _M)

    return gather


def kernel(x, table):
    d = table.shape[1]
    b_total = x.size
    assert b_total % (_NW * _C) == 0
    idx = x.reshape(_NW, b_total // (_NW * _C), _C).astype(jnp.int32)
    tab = _scale_table(table, math.sqrt(float(d)))
    out = _make_gather(b_total, d)(tab, idx)
    return out.reshape(x.shape + (d,))
